# Initial kernel scaffold; baseline (speedup 1.0000x reference)
import jax
import jax.numpy as jnp
from jax.experimental import pallas as pl


def kernel(B, h, w, d, x, row_w, col_w, dep_w):
    raise NotImplementedError("write your pallas kernel here")


# speedup vs baseline: 1.4230x; 1.4230x over previous
import functools

import jax
import jax.numpy as jnp
from jax import lax
from jax.experimental import pallas as pl
from jax.experimental.pallas import tpu as pltpu
from jax.experimental.pallas import tpu_sc as plsc

_H, _W, _D, _C = 16, 16, 64, 256
_L = 16
_NB = _H * _W * _D


def _body(xe, ye, ze, out, buf_a, buf_b, xrow, ybuf, zbuf, sem_a, sem_b):
    wid = lax.axis_index("s") * 2 + lax.axis_index("c")
    ih = wid // 2
    iwb = (wid % 2) * 8

    pltpu.sync_copy(xe.at[pl.ds(ih, 1)], xrow)
    pltpu.sync_copy(ye.at[pl.ds(iwb, 8)], ybuf)
    pltpu.sync_copy(ze, zbuf)

    def fill_x_section(buf):
        vecs = [xrow[0, pl.ds(c * _L, _L)] for c in range(_C // _L)]

        def body(r, carry):
            for c in range(_C // _L):
                buf[r, pl.ds(c * _L, _L)] = vecs[c]
            return carry

        lax.fori_loop(0, _D, body, 0)

    def fill_z_section(buf):
        def body(r, carry):
            for c in range(_C // _L):
                buf[r, pl.ds(2 * _C + c * _L, _L)] = zbuf[r, pl.ds(c * _L, _L)]
            return carry

        lax.fori_loop(0, _D, body, 0)

    def fill_y_section(buf, t):
        vecs = [ybuf[t, pl.ds(c * _L, _L)] for c in range(_C // _L)]

        def body(r, carry):
            for c in range(_C // _L):
                buf[r, pl.ds(_C + c * _L, _L)] = vecs[c]
            return carry

        lax.fori_loop(0, _D, body, 0)

    def out_copy(buf, t, b, sem):
        start = b * _NB + wid * 8 * _D + t * _D
        return pltpu.make_async_copy(buf, out.at[pl.ds(start, _D)], sem)

    fill_x_section(buf_a)
    fill_x_section(buf_b)
    fill_z_section(buf_a)
    fill_z_section(buf_b)

    for t in range(8):
        buf = buf_a if t % 2 == 0 else buf_b
        sem = sem_a if t % 2 == 0 else sem_b
        if t >= 2:
            for b in range(4):
                out_copy(buf, t - 2, b, sem).wait()
        fill_y_section(buf, t)
        for b in range(4):
            out_copy(buf, t, b, sem).start()
    for t in (6, 7):
        buf = buf_a if t % 2 == 0 else buf_b
        sem = sem_a if t % 2 == 0 else sem_b
        for b in range(4):
            out_copy(buf, t, b, sem).wait()


def kernel(B, h, w, d, x, row_w, col_w, dep_w):
    hs, c = row_w.shape
    ws = col_w.shape[0]
    ds_ = dep_w.shape[0]
    Bs = x.shape[0]
    i = (jnp.arange(hs) + 1) * (_H // hs) - 1 + (h - hs)
    j = (jnp.arange(ws) + 1) * (_W // ws) - 1 + (w - ws)
    k = (jnp.arange(ds_) + 1) * (_D // ds_) - 1 + (d - ds_)
    delta = (B - Bs) * jnp.float32(1.0)
    xe = jnp.take(row_w, i, axis=0) + delta
    ye = jnp.take(col_w, j, axis=0) + delta
    ze = jnp.take(dep_w, k, axis=0) + delta

    mesh = plsc.VectorSubcoreMesh(core_axis_name="c", subcore_axis_name="s")
    launch = functools.partial(
        pl.kernel,
        mesh=mesh,
        out_type=jax.ShapeDtypeStruct((Bs * _NB, 3 * _C), jnp.float32),
        scratch_types=[
            pltpu.VMEM((_D, 3 * _C), jnp.float32),
            pltpu.VMEM((_D, 3 * _C), jnp.float32),
            pltpu.VMEM((1, _C), jnp.float32),
            pltpu.VMEM((8, _C), jnp.float32),
            pltpu.VMEM((_D, _C), jnp.float32),
            pltpu.SemaphoreType.DMA,
            pltpu.SemaphoreType.DMA,
        ],
    )(_body)
    out = launch(xe, ye, ze)
    return out.reshape(Bs, _NB, 3 * c)
